# Initial kernel scaffold; baseline (speedup 1.0000x reference)
import jax
import jax.numpy as jnp
from jax.experimental import pallas as pl


def kernel(x, edge_index, W_down0, b_down0, W_down1, b_down1, W_down2, b_down2, p_pool1, p_pool2, W_up1, b_up1, W_up2, b_up2):
    raise NotImplementedError("write your pallas kernel here")


# speedup vs baseline: 2.3779x; 2.3779x over previous
import functools

import jax
import jax.numpy as jnp
from jax.experimental import pallas as pl
from jax.experimental.pallas import tpu as pltpu

_N = 10000
_E = 320000
_D = 128
_K1 = 5000
_K2 = 2500
_STD = 0.01
_SQRT_D = 0.1


def _prep_body(x_ref, w_ref, deg_ref, xw_ref, y_ref, dinv_ref):
    xw = jnp.dot(x_ref[...], w_ref[...], preferred_element_type=jnp.float32)
    dinv = jax.lax.rsqrt(deg_ref[...] + 2.0)
    xw_ref[...] = xw
    y_ref[...] = xw * dinv
    dinv_ref[...] = dinv


def _prep(x, w, deg):
    n = x.shape[0]
    return pl.pallas_call(
        _prep_body,
        out_shape=(
            jax.ShapeDtypeStruct((n, _D), jnp.float32),
            jax.ShapeDtypeStruct((n, _D), jnp.float32),
            jax.ShapeDtypeStruct((n, 1), jnp.float32),
        ),
    )(x, w, deg.reshape(n, 1))


def _post_body(acc_ref, xw_ref, dinv_ref, b_ref, o_ref, *, do_tanh):
    dinv = dinv_ref[...]
    o = acc_ref[...] * dinv + 2.0 * dinv * dinv * xw_ref[...] + b_ref[...]
    o_ref[...] = jnp.tanh(o) if do_tanh else o


def _post(acc, xw, dinv, b, do_tanh):
    n = acc.shape[0]
    return pl.pallas_call(
        functools.partial(_post_body, do_tanh=do_tanh),
        out_shape=jax.ShapeDtypeStruct((n, _D), jnp.float32),
    )(acc, xw, dinv, b.reshape(1, _D))


def _score_body(h_ref, p_ref, s_ref):
    p = p_ref[...]
    pn = p * jax.lax.rsqrt(jnp.sum(p * p))
    s_ref[...] = jnp.tanh(jnp.sum(h_ref[...] * pn, axis=1, keepdims=True))


def _score(h, p):
    n = h.shape[0]
    s = pl.pallas_call(
        _score_body,
        out_shape=jax.ShapeDtypeStruct((n, 1), jnp.float32),
    )(h, p.reshape(1, _D))
    return s[:, 0]


def _gcn(x, row, col, w_edge, W, b, n, do_tanh):
    deg = jnp.zeros((n,), jnp.float32).at[col].add(w_edge)
    xw, y, dinv = _prep(x, W, deg)
    acc = jnp.zeros((n, _D), jnp.float32).at[col].add(w_edge[:, None] * y[row])
    return _post(acc, xw, dinv, b, do_tanh)


def _pool(x, row, col, w_edge, score, k, n):
    vals, perm = jax.lax.top_k(score, k)
    xp = x[perm] * vals[:, None]
    inv = jnp.full((n,), -1, jnp.int32).at[perm].set(jnp.arange(k, dtype=jnp.int32))
    r = inv[row]
    c = inv[col]
    valid = (r >= 0) & (c >= 0)
    nrow = jnp.where(valid, r, 0)
    ncol = jnp.where(valid, c, 0)
    nw = jnp.where(valid, w_edge, 0.0)
    return xp, nrow, ncol, nw, perm


def kernel(x, edge_index, W_down0, b_down0, W_down1, b_down1, W_down2, b_down2,
           p_pool1, p_pool2, W_up1, b_up1, W_up2, b_up2):
    row = edge_index[0]
    col = edge_index[1]
    ew = jnp.ones((_E,), jnp.float32)

    h = _gcn(x, row, col, ew, W_down0, b_down0, _N, True)
    x0 = h
    s1 = _score(h, p_pool1)
    h, r1, c1, ew1, perm1 = _pool(h, row, col, ew, s1, _K1, _N)
    h = _gcn(h, r1, c1, ew1, W_down1, b_down1, _K1, True)
    x1 = h
    s2 = _score(h, p_pool2)
    h, r2, c2, ew2, perm2 = _pool(h, r1, c1, ew1, s2, _K2, _K1)
    h = _gcn(h, r2, c2, ew2, W_down2, b_down2, _K2, True)

    up = jnp.zeros_like(x1).at[perm2].set(h)
    h = x1 + up
    h = _gcn(h, r1, c1, ew1, W_up1, b_up1, _K1, True)
    up = jnp.zeros_like(x0).at[perm1].set(h)
    h = x0 + up
    drift = _gcn(h, row, col, ew, W_up2, b_up2, _N, False)

    z = jax.random.normal(jax.random.fold_in(jax.random.key(0), 777),
                          drift.shape, dtype=drift.dtype)
    return drift + _STD * z / _SQRT_D
